# Initial kernel scaffold; baseline (speedup 1.0000x reference)
import jax
import jax.numpy as jnp
from jax.experimental import pallas as pl


def kernel(x, edge_index, edge_attr, lin_in_W, lin_in_b, msg_W1, msg_b1, msg_W2, msg_b2, upd_W1, upd_b1, upd_W2, upd_b2, head_W1, head_b1, head_W2, head_b2):
    raise NotImplementedError("write your pallas kernel here")


# speedup vs baseline: 2.8444x; 2.8444x over previous
import functools

import jax
import jax.numpy as jnp
from jax import lax
from jax.experimental import pallas as pl
from jax.experimental.pallas import tpu as pltpu
from jax.experimental.pallas import tpu_sc as plsc

N = 10000
E = 320000
D = 128
ED = 4
H = 128
OUT = 3

NC = 2
NS = 16
NPAD = 10240
NW = NC * NS
EW = E // NW
C = 80
ITERS = EW // C

BE = 2560
BN = 2000

_sc_mesh = plsc.VectorSubcoreMesh(
    core_axis_name="c", subcore_axis_name="s", num_cores=NC, num_subcores=NS)


@functools.partial(
    pl.kernel,
    out_type=jax.ShapeDtypeStruct((E, H), jnp.float32),
    mesh=_sc_mesh,
    scratch_types=[
        pltpu.VMEM((ITERS, C), jnp.int32),
        pltpu.VMEM((C, H), jnp.float32),
        pltpu.SemaphoreType.DMA,
    ],
)
def _sc_gather(table_hbm, idx_hbm, out_hbm, idx_v, rows_v, sem):
    c = lax.axis_index("c")
    s = lax.axis_index("s")
    wid = c * NS + s
    pltpu.sync_copy(idx_hbm.at[wid], idx_v)
    base = wid * EW

    def body(j, carry):
        pltpu.async_copy(table_hbm.at[idx_v.at[j]], rows_v, sem).wait()
        pltpu.sync_copy(rows_v, out_hbm.at[pl.ds(base + j * C, C)])
        return carry

    lax.fori_loop(0, ITERS, body, 0)


@functools.partial(
    pl.kernel,
    out_type=jax.ShapeDtypeStruct((NC, NPAD, H), jnp.float32),
    mesh=_sc_mesh,
    scratch_types=[
        pltpu.VMEM((ITERS, C), jnp.int32),
        pltpu.VMEM((C, H), jnp.float32),
        pltpu.VMEM_SHARED((NPAD, H), jnp.float32),
        pltpu.SemaphoreType.DMA,
    ],
)
def _sc_scatter(m_hbm, idx_hbm, zeros_hbm, out_hbm, idx_v, rows_v, acc_sh, sem):
    c = lax.axis_index("c")
    s = lax.axis_index("s")
    wid = c * NS + s
    rows_per_tile = NPAD // NS
    pltpu.sync_copy(zeros_hbm.at[pl.ds(s * rows_per_tile, rows_per_tile)],
                    acc_sh.at[pl.ds(s * rows_per_tile, rows_per_tile)])
    plsc.subcore_barrier()

    pltpu.sync_copy(idx_hbm.at[wid], idx_v)
    base = wid * EW

    def body(j, carry):
        pltpu.async_copy(m_hbm.at[pl.ds(base + j * C, C)], rows_v, sem).wait()
        pltpu.sync_copy(rows_v, acc_sh.at[idx_v.at[j]], add=True)
        return carry

    lax.fori_loop(0, ITERS, body, 0)
    plsc.subcore_barrier()
    pltpu.sync_copy(acc_sh.at[pl.ds(s * rows_per_tile, rows_per_tile)],
                    out_hbm.at[c, pl.ds(s * rows_per_tile, rows_per_tile)])


@functools.partial(
    pl.kernel,
    out_type=jax.ShapeDtypeStruct((NC, NPAD, H), jnp.float32),
    mesh=_sc_mesh,
    scratch_types=[
        pltpu.VMEM((ITERS, C), jnp.int32),
        pltpu.VMEM((C, H), jnp.float32),
        pltpu.VMEM_SHARED((NPAD, H), jnp.float32),
        pltpu.SemaphoreType.DMA,
    ],
)
def _sc_degree(idx_hbm, ones_hbm, zeros_hbm, out_hbm, idx_v, ones_v, cnt_sh,
               sem):
    c = lax.axis_index("c")
    s = lax.axis_index("s")
    wid = c * NS + s
    rows_per_tile = NPAD // NS
    pltpu.sync_copy(zeros_hbm.at[pl.ds(s * rows_per_tile, rows_per_tile)],
                    cnt_sh.at[pl.ds(s * rows_per_tile, rows_per_tile)])
    pltpu.sync_copy(ones_hbm, ones_v)
    plsc.subcore_barrier()

    pltpu.sync_copy(idx_hbm.at[wid], idx_v)

    def body(j, carry):
        pltpu.sync_copy(ones_v, cnt_sh.at[idx_v.at[j]], add=True)
        return carry

    lax.fori_loop(0, ITERS, body, 0)
    plsc.subcore_barrier()
    pltpu.sync_copy(cnt_sh.at[pl.ds(s * rows_per_tile, rows_per_tile)],
                    out_hbm.at[c, pl.ds(s * rows_per_tile, rows_per_tile)])


def _lin_in_body(x_ref, w_ref, b_ref, o_ref):
    acc = jnp.dot(x_ref[...], w_ref[...], preferred_element_type=jnp.float32)
    o_ref[...] = jnp.maximum(acc + b_ref[...], 0.0)


def _lin_in(x, w, b):
    return pl.pallas_call(
        _lin_in_body,
        grid=(N // BN,),
        in_specs=[
            pl.BlockSpec((BN, D), lambda i: (i, 0)),
            pl.BlockSpec((D, H), lambda i: (0, 0)),
            pl.BlockSpec((1, H), lambda i: (0, 0)),
        ],
        out_specs=pl.BlockSpec((BN, H), lambda i: (i, 0)),
        out_shape=jax.ShapeDtypeStruct((N, H), jnp.float32),
    )(x, w, b.reshape(1, H))


def _edge_mlp_body(xj_ref, ea_ref, w1x_ref, w1e_ref, b1_ref, w2_ref, b2_ref,
                   o_ref):
    m1 = jnp.dot(xj_ref[...], w1x_ref[...], preferred_element_type=jnp.float32)
    m1 += jnp.dot(ea_ref[...], w1e_ref[...], preferred_element_type=jnp.float32)
    m1 = jnp.maximum(m1 + b1_ref[...], 0.0)
    m2 = jnp.dot(m1, w2_ref[...], preferred_element_type=jnp.float32)
    o_ref[...] = jnp.maximum(m2 + b2_ref[...], 0.0)


def _edge_mlp(xj, ea, w1x, w1e, b1, w2, b2):
    return pl.pallas_call(
        _edge_mlp_body,
        grid=(E // BE,),
        in_specs=[
            pl.BlockSpec((BE, H), lambda i: (i, 0)),
            pl.BlockSpec((BE, ED), lambda i: (i, 0)),
            pl.BlockSpec((H, H), lambda i: (0, 0)),
            pl.BlockSpec((ED, H), lambda i: (0, 0)),
            pl.BlockSpec((1, H), lambda i: (0, 0)),
            pl.BlockSpec((H, H), lambda i: (0, 0)),
            pl.BlockSpec((1, H), lambda i: (0, 0)),
        ],
        out_specs=pl.BlockSpec((BE, H), lambda i: (i, 0)),
        out_shape=jax.ShapeDtypeStruct((E, H), jnp.float32),
    )(xj, ea, w1x, w1e, b1.reshape(1, H), w2, b2.reshape(1, H))


def _update_body(h_ref, p0_ref, p1_ref, c0_ref, c1_ref, u1h_ref, u1a_ref,
                 ub1_ref, u2_ref, ub2_ref, o_ref):
    deg = jnp.maximum(c0_ref[:, :1] + c1_ref[:, :1], 1.0)
    agg = (p0_ref[...] + p1_ref[...]) / deg
    h = h_ref[...]
    u = jnp.dot(h, u1h_ref[...], preferred_element_type=jnp.float32)
    u += jnp.dot(agg, u1a_ref[...], preferred_element_type=jnp.float32)
    u = jnp.maximum(u + ub1_ref[...], 0.0)
    u2 = jnp.dot(u, u2_ref[...], preferred_element_type=jnp.float32)
    o_ref[...] = jnp.maximum(u2 + ub2_ref[...] + h, 0.0)


def _update(h, p0, p1, c0, c1, u1h, u1a, ub1, u2, ub2):
    return pl.pallas_call(
        _update_body,
        grid=(N // BN,),
        in_specs=[
            pl.BlockSpec((BN, H), lambda i: (i, 0)),
            pl.BlockSpec((BN, H), lambda i: (i, 0)),
            pl.BlockSpec((BN, H), lambda i: (i, 0)),
            pl.BlockSpec((BN, H), lambda i: (i, 0)),
            pl.BlockSpec((BN, H), lambda i: (i, 0)),
            pl.BlockSpec((H, H), lambda i: (0, 0)),
            pl.BlockSpec((H, H), lambda i: (0, 0)),
            pl.BlockSpec((1, H), lambda i: (0, 0)),
            pl.BlockSpec((H, H), lambda i: (0, 0)),
            pl.BlockSpec((1, H), lambda i: (0, 0)),
        ],
        out_specs=pl.BlockSpec((BN, H), lambda i: (i, 0)),
        out_shape=jax.ShapeDtypeStruct((N, H), jnp.float32),
    )(h, p0, p1, c0, c1, u1h, u1a, ub1.reshape(1, H), u2, ub2.reshape(1, H))


def _head_body(h_ref, w1_ref, b1_ref, w2_ref, b2_ref, o_ref):
    t = jnp.dot(h_ref[...], w1_ref[...], preferred_element_type=jnp.float32)
    t = jnp.maximum(t + b1_ref[...], 0.0)
    o = jnp.dot(t, w2_ref[...], preferred_element_type=jnp.float32)
    o_ref[...] = jnp.maximum(o + b2_ref[...], 0.0)


def _head(h, w1, b1, w2, b2):
    return pl.pallas_call(
        _head_body,
        grid=(N // BN,),
        in_specs=[
            pl.BlockSpec((BN, H), lambda i: (i, 0)),
            pl.BlockSpec((H, H), lambda i: (0, 0)),
            pl.BlockSpec((1, H), lambda i: (0, 0)),
            pl.BlockSpec((H, OUT), lambda i: (0, 0)),
            pl.BlockSpec((1, OUT), lambda i: (0, 0)),
        ],
        out_specs=pl.BlockSpec((BN, OUT), lambda i: (i, 0)),
        out_shape=jax.ShapeDtypeStruct((N, OUT), jnp.float32),
    )(h, w1, b1.reshape(1, H), w2, b2.reshape(1, OUT))


def kernel(x, edge_index, edge_attr, lin_in_W, lin_in_b, msg_W1, msg_b1,
           msg_W2, msg_b2, upd_W1, upd_b1, upd_W2, upd_b2, head_W1, head_b1,
           head_W2, head_b2):
    src3 = edge_index[0].reshape(NW, ITERS, C)
    dst3 = edge_index[1].reshape(NW, ITERS, C)
    zeros_nh = jnp.zeros((NPAD, H), jnp.float32)
    ones_ch = jnp.ones((C, H), jnp.float32)

    cnt = _sc_degree(dst3, ones_ch, zeros_nh)
    h = _lin_in(x, lin_in_W, lin_in_b)
    L = msg_W1.shape[0]
    for l in range(L):
        xj = _sc_gather(h, src3)
        m = _edge_mlp(xj, edge_attr, msg_W1[l][:H], msg_W1[l][H:],
                      msg_b1[l], msg_W2[l], msg_b2[l])
        p = _sc_scatter(m, dst3, zeros_nh)
        h = _update(h, p[0], p[1], cnt[0], cnt[1],
                    upd_W1[l][:H], upd_W1[l][H:], upd_b1[l],
                    upd_W2[l], upd_b2[l])
    return _head(h, head_W1, head_b1, head_W2, head_b2)
